# baseline (device time: 121973 ns/iter reference)
import jax
import jax.numpy as jnp
from jax import lax
from jax.experimental import pallas as pl
from jax.experimental.pallas import tpu as pltpu

N_DEV = 4


def kernel(x, Win0, Wout0, Win1, Wout1, Win2, Wout2):
    m_per, d = x.shape
    _, h_per = Win0.shape

    def body(x_ref, win0, wout0, win1, wout1, win2, wout2, out_ref,
             X4, P4, R, ag_send, ag_recv, rs_send, rs_recv):
        me = lax.axis_index("i")
        left = lax.rem(me - 1 + N_DEV, N_DEV)
        right = lax.rem(me + 1, N_DEV)

        barrier = pltpu.get_barrier_semaphore()
        for nbr in (left, right):
            pl.semaphore_signal(
                barrier, inc=1,
                device_id=(nbr,), device_id_type=pl.DeviceIdType.MESH,
            )
        pl.semaphore_wait(barrier, 2)

        X4[0] = x_ref[...].astype(jnp.bfloat16)

        for l, (win, wout) in enumerate(
            [(win0, wout0), (win1, wout1), (win2, wout2)]
        ):
            for h in range(N_DEV - 1):
                rdma = pltpu.make_async_remote_copy(
                    src_ref=X4.at[h],
                    dst_ref=X4.at[h + 1],
                    send_sem=ag_send.at[h],
                    recv_sem=ag_recv.at[h],
                    device_id=(right,),
                    device_id_type=pl.DeviceIdType.MESH,
                )
                rdma.start()
                rdma.wait()

            wb = win[...].astype(jnp.bfloat16)
            wob = wout[...].astype(jnp.bfloat16)
            xa = X4[...].reshape(N_DEV * m_per, d)
            h1 = jnp.maximum(
                jnp.dot(xa, wb, preferred_element_type=jnp.float32), 0.0
            ).astype(jnp.bfloat16)
            p = jnp.dot(h1, wob, preferred_element_type=jnp.float32)
            P4[...] = p.reshape(N_DEV, m_per, d)

            r1 = pltpu.make_async_remote_copy(
                src_ref=P4.at[1], dst_ref=R.at[2],
                send_sem=rs_send.at[0], recv_sem=rs_recv.at[0],
                device_id=(right,), device_id_type=pl.DeviceIdType.MESH,
            )
            r1.start()
            r1.wait()
            R[2] = R[2] + P4[2]

            r2 = pltpu.make_async_remote_copy(
                src_ref=R.at[2], dst_ref=R.at[3],
                send_sem=rs_send.at[1], recv_sem=rs_recv.at[1],
                device_id=(right,), device_id_type=pl.DeviceIdType.MESH,
            )
            r2.start()
            r2.wait()
            R[3] = R[3] + P4[3]

            r3 = pltpu.make_async_remote_copy(
                src_ref=R.at[3], dst_ref=R.at[0],
                send_sem=rs_send.at[2], recv_sem=rs_recv.at[2],
                device_id=(right,), device_id_type=pl.DeviceIdType.MESH,
            )
            r3.start()
            r3.wait()
            res = R[0] + P4[0]

            if l < 2:
                X4[0] = res.astype(jnp.bfloat16)
            else:
                out_ref[...] = res

    return pl.pallas_call(
        body,
        out_shape=jax.ShapeDtypeStruct((m_per, d), jnp.float32),
        in_specs=[pl.BlockSpec(memory_space=pltpu.VMEM)] * 7,
        out_specs=pl.BlockSpec(memory_space=pltpu.VMEM),
        scratch_shapes=[
            pltpu.VMEM((N_DEV, m_per, d), jnp.bfloat16),
            pltpu.VMEM((N_DEV, m_per, d), jnp.float32),
            pltpu.VMEM((N_DEV, m_per, d), jnp.float32),
            pltpu.SemaphoreType.DMA((N_DEV - 1,)),
            pltpu.SemaphoreType.DMA((N_DEV - 1,)),
            pltpu.SemaphoreType.DMA((N_DEV - 1,)),
            pltpu.SemaphoreType.DMA((N_DEV - 1,)),
        ],
        compiler_params=pltpu.CompilerParams(collective_id=0),
    )(x, Win0, Wout0, Win1, Wout1, Win2, Wout2)


# device time: 84791 ns/iter; 1.4385x vs baseline; 1.4385x over previous
import jax
import jax.numpy as jnp
from jax import lax
from jax.experimental import pallas as pl
from jax.experimental.pallas import tpu as pltpu

N_DEV = 4
N_RDMA = 18


def kernel(x, Win0, Wout0, Win1, Wout1, Win2, Wout2):
    m_per, d = x.shape
    _, h_per = Win0.shape

    def body(x_ref, win0, wout0, win1, wout1, win2, wout2, out_ref,
             X4, P4, RS, FB, ST, send_sems, recv_sems):
        me = lax.axis_index("i")
        ypart = me ^ 1
        xpart = 3 - me

        barrier = pltpu.get_barrier_semaphore()
        for nbr in (ypart, xpart):
            pl.semaphore_signal(
                barrier, inc=1,
                device_id=(nbr,), device_id_type=pl.DeviceIdType.MESH,
            )
        pl.semaphore_wait(barrier, 2)

        idx4 = lax.broadcasted_iota(jnp.int32, (N_DEV, m_per, d), 0)
        sel_mine = (idx4 == me) | (idx4 == xpart)
        sem = iter(range(N_RDMA))

        def exch(src, dst, blocks, partner):
            rdmas = []
            for b in blocks:
                i = next(sem)
                r = pltpu.make_async_remote_copy(
                    src_ref=src.at[b],
                    dst_ref=dst.at[b],
                    send_sem=send_sems.at[i],
                    recv_sem=recv_sems.at[i],
                    device_id=(partner,),
                    device_id_type=pl.DeviceIdType.MESH,
                )
                r.start()
                rdmas.append(r)
            for r in rdmas:
                r.wait()

        xb = x_ref[...].astype(jnp.bfloat16)[None]
        X4[...] = jnp.where(idx4 == me, xb, jnp.bfloat16(0))
        exch(X4, X4, [me], xpart)
        exch(X4, X4, [me, xpart], ypart)

        for l, (win, wout) in enumerate(
            [(win0, wout0), (win1, wout1), (win2, wout2)]
        ):
            wb = win[...].astype(jnp.bfloat16)
            wob = wout[...].astype(jnp.bfloat16)
            xa = X4[...].reshape(N_DEV * m_per, d)
            h1 = jnp.maximum(
                jnp.dot(xa, wb, preferred_element_type=jnp.float32), 0.0
            ).astype(jnp.bfloat16)
            p = jnp.dot(h1, wob, preferred_element_type=jnp.float32)
            P4[...] = p.reshape(N_DEV, m_per, d)

            ST[...] = P4[...].astype(jnp.bfloat16)
            exch(ST, RS, [ypart, me ^ 2], ypart)
            P4[...] = P4[...] + jnp.where(
                sel_mine, RS[...].astype(jnp.float32), 0.0
            )

            if l < 2:
                ST[...] = P4[...].astype(jnp.bfloat16)
                exch(ST, FB, [me, xpart], xpart)
                P4[...] = P4[...] + jnp.where(
                    sel_mine, FB[...].astype(jnp.float32), 0.0
                )
                X4[...] = jnp.where(
                    sel_mine, P4[...].astype(jnp.bfloat16), X4[...]
                )
                exch(X4, X4, [me, xpart], ypart)
            else:
                ST[...] = P4[...].astype(jnp.bfloat16)
                exch(ST, FB, [xpart], xpart)
                out_ref[...] = jnp.sum(
                    jnp.where(
                        idx4 == me,
                        P4[...] + FB[...].astype(jnp.float32),
                        0.0,
                    ),
                    axis=0,
                )

    return pl.pallas_call(
        body,
        out_shape=jax.ShapeDtypeStruct((m_per, d), jnp.float32),
        in_specs=[pl.BlockSpec(memory_space=pltpu.VMEM)] * 7,
        out_specs=pl.BlockSpec(memory_space=pltpu.VMEM),
        scratch_shapes=[
            pltpu.VMEM((N_DEV, m_per, d), jnp.bfloat16),
            pltpu.VMEM((N_DEV, m_per, d), jnp.float32),
            pltpu.VMEM((N_DEV, m_per, d), jnp.bfloat16),
            pltpu.VMEM((N_DEV, m_per, d), jnp.bfloat16),
            pltpu.VMEM((N_DEV, m_per, d), jnp.bfloat16),
            pltpu.SemaphoreType.DMA((N_RDMA,)),
            pltpu.SemaphoreType.DMA((N_RDMA,)),
        ],
        compiler_params=pltpu.CompilerParams(collective_id=0),
    )(x, Win0, Wout0, Win1, Wout1, Win2, Wout2)


# device time: 60351 ns/iter; 2.0211x vs baseline; 1.4050x over previous
import jax
import jax.numpy as jnp
from jax import lax
from jax.experimental import pallas as pl
from jax.experimental.pallas import tpu as pltpu

N_DEV = 4
N_RDMA = 36


def kernel(x, Win0, Wout0, Win1, Wout1, Win2, Wout2):
    m_per, d = x.shape
    dh = d // 2
    _, h_per = Win0.shape

    def body(x_ref, win0, wout0, win1, wout1, win2, wout2, out_ref,
             X4A, X4B, P4, RSA, RSB, FBA, FBB, STA, STB,
             send_sems, recv_sems):
        me = lax.axis_index("i")
        ypart = me ^ 1
        xpart = 3 - me

        barrier = pltpu.get_barrier_semaphore()
        for nbr in (ypart, xpart):
            pl.semaphore_signal(
                barrier, inc=1,
                device_id=(nbr,), device_id_type=pl.DeviceIdType.MESH,
            )
        pl.semaphore_wait(barrier, 2)

        idx = lax.broadcasted_iota(jnp.int32, (N_DEV, m_per, dh), 0)
        idxf = lax.broadcasted_iota(jnp.int32, (N_DEV, m_per, d), 0)
        selA = (idx == me) | (idx == xpart)
        selB = (idx == me) | (idx == ypart)
        sem = iter(range(N_RDMA))

        def exch(*quads):
            rdmas = []
            for src, dst, b, partner in quads:
                i = next(sem)
                r = pltpu.make_async_remote_copy(
                    src_ref=src.at[b],
                    dst_ref=dst.at[b],
                    send_sem=send_sems.at[i],
                    recv_sem=recv_sems.at[i],
                    device_id=(partner,),
                    device_id_type=pl.DeviceIdType.MESH,
                )
                r.start()
                rdmas.append(r)
            for r in rdmas:
                r.wait()

        def stage():
            pv = P4[...]
            STA[...] = pv[:, :, :dh].astype(jnp.bfloat16)
            STB[...] = pv[:, :, dh:].astype(jnp.bfloat16)

        def acc(bufA, bufB):
            P4[...] = P4[...] + jnp.concatenate(
                [
                    jnp.where(selA, bufA[...].astype(jnp.float32), 0.0),
                    jnp.where(selB, bufB[...].astype(jnp.float32), 0.0),
                ],
                axis=2,
            )

        xb = x_ref[...].astype(jnp.bfloat16)
        X4A[...] = jnp.where(idx == me, xb[None, :, :dh], jnp.bfloat16(0))
        X4B[...] = jnp.where(idx == me, xb[None, :, dh:], jnp.bfloat16(0))
        exch((X4A, X4A, me, xpart), (X4B, X4B, me, ypart))
        exch(
            (X4A, X4A, me, ypart), (X4A, X4A, xpart, ypart),
            (X4B, X4B, me, xpart), (X4B, X4B, ypart, xpart),
        )

        for l, (win, wout) in enumerate(
            [(win0, wout0), (win1, wout1), (win2, wout2)]
        ):
            wb = win[...].astype(jnp.bfloat16)
            wob = wout[...].astype(jnp.bfloat16)
            xa = jnp.concatenate(
                [
                    X4A[...].reshape(N_DEV * m_per, dh),
                    X4B[...].reshape(N_DEV * m_per, dh),
                ],
                axis=1,
            )
            h1 = jnp.maximum(
                jnp.dot(xa, wb, preferred_element_type=jnp.float32), 0.0
            ).astype(jnp.bfloat16)
            p = jnp.dot(h1, wob, preferred_element_type=jnp.float32)
            P4[...] = p.reshape(N_DEV, m_per, d)

            stage()
            exch(
                (STA, RSA, ypart, ypart), (STA, RSA, me ^ 2, ypart),
                (STB, RSB, xpart, xpart), (STB, RSB, (3 - me) ^ 1, xpart),
            )
            acc(RSA, RSB)

            if l < 2:
                stage()
                exch(
                    (STA, FBA, me, xpart), (STA, FBA, xpart, xpart),
                    (STB, FBB, me, ypart), (STB, FBB, ypart, ypart),
                )
                acc(FBA, FBB)
                pv = P4[...]
                X4A[...] = jnp.where(
                    selA, pv[:, :, :dh].astype(jnp.bfloat16), X4A[...]
                )
                X4B[...] = jnp.where(
                    selB, pv[:, :, dh:].astype(jnp.bfloat16), X4B[...]
                )
                exch(
                    (X4A, X4A, me, ypart), (X4A, X4A, xpart, ypart),
                    (X4B, X4B, me, xpart), (X4B, X4B, ypart, xpart),
                )
            else:
                stage()
                exch(
                    (STA, FBA, xpart, xpart), (STB, FBB, ypart, ypart),
                )
                fb = jnp.concatenate(
                    [
                        FBA[...].astype(jnp.float32),
                        FBB[...].astype(jnp.float32),
                    ],
                    axis=2,
                )
                out_ref[...] = jnp.sum(
                    jnp.where(idxf == me, P4[...] + fb, 0.0), axis=0
                )

    return pl.pallas_call(
        body,
        out_shape=jax.ShapeDtypeStruct((m_per, d), jnp.float32),
        in_specs=[pl.BlockSpec(memory_space=pltpu.VMEM)] * 7,
        out_specs=pl.BlockSpec(memory_space=pltpu.VMEM),
        scratch_shapes=[
            pltpu.VMEM((N_DEV, m_per, dh), jnp.bfloat16),
            pltpu.VMEM((N_DEV, m_per, dh), jnp.bfloat16),
            pltpu.VMEM((N_DEV, m_per, d), jnp.float32),
            pltpu.VMEM((N_DEV, m_per, dh), jnp.bfloat16),
            pltpu.VMEM((N_DEV, m_per, dh), jnp.bfloat16),
            pltpu.VMEM((N_DEV, m_per, dh), jnp.bfloat16),
            pltpu.VMEM((N_DEV, m_per, dh), jnp.bfloat16),
            pltpu.VMEM((N_DEV, m_per, dh), jnp.bfloat16),
            pltpu.VMEM((N_DEV, m_per, dh), jnp.bfloat16),
            pltpu.SemaphoreType.DMA((N_RDMA,)),
            pltpu.SemaphoreType.DMA((N_RDMA,)),
        ],
        compiler_params=pltpu.CompilerParams(collective_id=0),
    )(x, Win0, Wout0, Win1, Wout1, Win2, Wout2)


# device time: 60171 ns/iter; 2.0271x vs baseline; 1.0030x over previous
import jax
import jax.numpy as jnp
from jax import lax
from jax.experimental import pallas as pl
from jax.experimental.pallas import tpu as pltpu

N_DEV = 4
N_RDMA = 36


def kernel(x, Win0, Wout0, Win1, Wout1, Win2, Wout2):
    m_per, d = x.shape
    dh = d // 2
    _, h_per = Win0.shape

    def body(x_ref, win0, wout0, win1, wout1, win2, wout2, out_ref,
             X4A, X4B, PA, PB, RSA, RSB, FBA, FBB,
             send_sems, recv_sems):
        me = lax.axis_index("i")
        ypart = me ^ 1
        xpart = 3 - me

        idx = lax.broadcasted_iota(jnp.int32, (N_DEV, m_per, dh), 0)
        selA = (idx == me) | (idx == xpart)
        selB = (idx == me) | (idx == ypart)
        sem = iter(range(N_RDMA))
        zb = jnp.bfloat16(0)

        def start(*quads):
            rdmas = []
            for src, dst, b, partner in quads:
                i = next(sem)
                r = pltpu.make_async_remote_copy(
                    src_ref=src.at[b],
                    dst_ref=dst.at[b],
                    send_sem=send_sems.at[i],
                    recv_sem=recv_sems.at[i],
                    device_id=(partner,),
                    device_id_type=pl.DeviceIdType.MESH,
                )
                r.start()
                rdmas.append(r)
            return rdmas

        def wait(rdmas):
            for r in rdmas:
                r.wait()

        xb = x_ref[...].astype(jnp.bfloat16)
        X4A[...] = jnp.where(idx == me, xb[None, :, :dh], zb)
        X4B[...] = jnp.where(idx == me, xb[None, :, dh:], zb)

        barrier = pltpu.get_barrier_semaphore()
        for nbr in (ypart, xpart):
            pl.semaphore_signal(
                barrier, inc=1,
                device_id=(nbr,), device_id_type=pl.DeviceIdType.MESH,
            )
        pl.semaphore_wait(barrier, 2)

        r = start((X4A, X4A, me, xpart), (X4B, X4B, me, ypart))
        wbs = [
            (w[...].astype(jnp.bfloat16), wo[...].astype(jnp.bfloat16))
            for w, wo in [(win0, wout0), (win1, wout1), (win2, wout2)]
        ]
        wait(r)
        wait(start(
            (X4A, X4A, me, ypart), (X4A, X4A, xpart, ypart),
            (X4B, X4B, me, xpart), (X4B, X4B, ypart, xpart),
        ))

        for l, (wb, wob) in enumerate(wbs):
            if l == 0:
                mA, mB = X4A[...], X4B[...]
            else:
                mA = jnp.where(selA, PA[...], X4A[...])
                mB = jnp.where(selB, PB[...], X4B[...])
            xa = jnp.concatenate(
                [
                    mA.reshape(N_DEV * m_per, dh),
                    mB.reshape(N_DEV * m_per, dh),
                ],
                axis=1,
            )
            h1 = jnp.maximum(
                jnp.dot(xa, wb, preferred_element_type=jnp.float32), 0.0
            ).astype(jnp.bfloat16)
            p = jnp.dot(h1, wob, preferred_element_type=jnp.float32)
            pb16 = p.astype(jnp.bfloat16)
            PA[...] = pb16[:, :dh].reshape(N_DEV, m_per, dh)
            PB[...] = pb16[:, dh:].reshape(N_DEV, m_per, dh)

            wait(start(
                (PA, RSA, ypart, ypart), (PA, RSA, me ^ 2, ypart),
                (PB, RSB, xpart, xpart), (PB, RSB, (3 - me) ^ 1, xpart),
            ))
            PA[...] = PA[...] + jnp.where(selA, RSA[...], zb)
            PB[...] = PB[...] + jnp.where(selB, RSB[...], zb)

            if l < 2:
                wait(start(
                    (PA, FBA, me, xpart), (PA, FBA, xpart, xpart),
                    (PB, FBB, me, ypart), (PB, FBB, ypart, ypart),
                ))
                PA[...] = PA[...] + jnp.where(selA, FBA[...], zb)
                PB[...] = PB[...] + jnp.where(selB, FBB[...], zb)
                wait(start(
                    (PA, X4A, me, ypart), (PA, X4A, xpart, ypart),
                    (PB, X4B, me, xpart), (PB, X4B, ypart, xpart),
                ))
            else:
                wait(start(
                    (PA, FBA, xpart, xpart), (PB, FBB, ypart, ypart),
                ))
                outA = jnp.sum(
                    jnp.where(
                        idx == me,
                        PA[...].astype(jnp.float32)
                        + FBA[...].astype(jnp.float32),
                        0.0,
                    ),
                    axis=0,
                )
                outB = jnp.sum(
                    jnp.where(
                        idx == me,
                        PB[...].astype(jnp.float32)
                        + FBB[...].astype(jnp.float32),
                        0.0,
                    ),
                    axis=0,
                )
                out_ref[...] = jnp.concatenate([outA, outB], axis=1)

    return pl.pallas_call(
        body,
        out_shape=jax.ShapeDtypeStruct((m_per, d), jnp.float32),
        in_specs=[pl.BlockSpec(memory_space=pltpu.VMEM)] * 7,
        out_specs=pl.BlockSpec(memory_space=pltpu.VMEM),
        scratch_shapes=[
            pltpu.VMEM((N_DEV, m_per, dh), jnp.bfloat16),
            pltpu.VMEM((N_DEV, m_per, dh), jnp.bfloat16),
            pltpu.VMEM((N_DEV, m_per, dh), jnp.bfloat16),
            pltpu.VMEM((N_DEV, m_per, dh), jnp.bfloat16),
            pltpu.VMEM((N_DEV, m_per, dh), jnp.bfloat16),
            pltpu.VMEM((N_DEV, m_per, dh), jnp.bfloat16),
            pltpu.VMEM((N_DEV, m_per, dh), jnp.bfloat16),
            pltpu.VMEM((N_DEV, m_per, dh), jnp.bfloat16),
            pltpu.SemaphoreType.DMA((N_RDMA,)),
            pltpu.SemaphoreType.DMA((N_RDMA,)),
        ],
        compiler_params=pltpu.CompilerParams(collective_id=0),
    )(x, Win0, Wout0, Win1, Wout1, Win2, Wout2)


# device time: 41062 ns/iter; 2.9705x vs baseline; 1.4654x over previous
import jax
import jax.numpy as jnp
from jax import lax
from jax.experimental import pallas as pl
from jax.experimental.pallas import tpu as pltpu

N_DEV = 4
N_RDMA = 18


def kernel(x, Win0, Wout0, Win1, Wout1, Win2, Wout2):
    m_per, d = x.shape
    _, h_per = Win0.shape

    def body(x_ref, win0, wout0, win1, wout1, win2, wout2, out_ref,
             W0, W1, W2, V0, V1, V2, send_sems, recv_sems):
        me = lax.axis_index("i")
        ypart = me ^ 1
        xpart = 3 - me

        idxW = lax.broadcasted_iota(jnp.int32, (N_DEV, d, h_per), 0)
        idxV = lax.broadcasted_iota(jnp.int32, (N_DEV, h_per, d), 0)
        sem = iter(range(N_RDMA))

        def start(*quads):
            rdmas = []
            for src, dst, b, partner in quads:
                i = next(sem)
                r = pltpu.make_async_remote_copy(
                    src_ref=src.at[b],
                    dst_ref=dst.at[b],
                    send_sem=send_sems.at[i],
                    recv_sem=recv_sems.at[i],
                    device_id=(partner,),
                    device_id_type=pl.DeviceIdType.MESH,
                )
                r.start()
                rdmas.append(r)
            return rdmas

        def wait(rdmas):
            for r in rdmas:
                r.wait()

        zb = jnp.bfloat16(0)
        for ref, src_ref, idx in (
            (W0, win0, idxW), (W1, win1, idxW), (W2, win2, idxW),
            (V0, wout0, idxV), (V1, wout1, idxV), (V2, wout2, idxV),
        ):
            ref[...] = jnp.where(
                idx == me, src_ref[...].astype(jnp.bfloat16)[None], zb
            )

        barrier = pltpu.get_barrier_semaphore()
        for nbr in (ypart, xpart):
            pl.semaphore_signal(
                barrier, inc=1,
                device_id=(nbr,), device_id_type=pl.DeviceIdType.MESH,
            )
        pl.semaphore_wait(barrier, 2)

        r1 = [
            start((W, W, me, ypart), (V, V, me, xpart))
            for W, V in ((W0, V0), (W1, V1), (W2, V2))
        ]
        r2 = []
        for (W, V), r in zip(((W0, V0), (W1, V1), (W2, V2)), r1):
            wait(r)
            r2.append(start(
                (W, W, me, xpart), (W, W, ypart, xpart),
                (V, V, me, ypart), (V, V, xpart, ypart),
            ))

        xl = x_ref[...].astype(jnp.bfloat16)
        for l, (W, V) in enumerate(((W0, V0), (W1, V1), (W2, V2))):
            wait(r2[l])
            acc = jnp.zeros((m_per, d), jnp.float32)
            for q in range(N_DEV):
                hq = jnp.maximum(
                    jnp.dot(xl, W[q], preferred_element_type=jnp.float32),
                    0.0,
                ).astype(jnp.bfloat16)
                acc = acc + jnp.dot(
                    hq, V[q], preferred_element_type=jnp.float32
                )
            if l < 2:
                xl = acc.astype(jnp.bfloat16)
            else:
                out_ref[...] = acc

    return pl.pallas_call(
        body,
        out_shape=jax.ShapeDtypeStruct((m_per, d), jnp.float32),
        in_specs=[pl.BlockSpec(memory_space=pltpu.VMEM)] * 7,
        out_specs=pl.BlockSpec(memory_space=pltpu.VMEM),
        scratch_shapes=[
            pltpu.VMEM((N_DEV, d, h_per), jnp.bfloat16),
            pltpu.VMEM((N_DEV, d, h_per), jnp.bfloat16),
            pltpu.VMEM((N_DEV, d, h_per), jnp.bfloat16),
            pltpu.VMEM((N_DEV, h_per, d), jnp.bfloat16),
            pltpu.VMEM((N_DEV, h_per, d), jnp.bfloat16),
            pltpu.VMEM((N_DEV, h_per, d), jnp.bfloat16),
            pltpu.SemaphoreType.DMA((N_RDMA,)),
            pltpu.SemaphoreType.DMA((N_RDMA,)),
        ],
        compiler_params=pltpu.CompilerParams(collective_id=0),
    )(x, Win0, Wout0, Win1, Wout1, Win2, Wout2)
